# baseline (speedup 1.0000x reference)
import functools

import jax
import jax.numpy as jnp
from jax import lax
from jax.experimental import pallas as pl
from jax.experimental.pallas import tpu as pltpu
from jax.experimental.pallas import tpu_sc as plsc

BATCH = 16384
N_FIELDS = 100
LANES = 16
NW = 32
NS = 16
B_PER_W = BATCH // NW
N_PASS = 4
B_Q = B_PER_W // N_PASS
IDX_Q = B_Q * N_FIELDS
TABLE = 1000001
SPM_TABLE = 1000448
W_CHUNK = 62592
LAST_BASE = 15 * W_CHUNK
PIECE = 7808
TAIL_BASE = TABLE - TABLE % 128


def _pieces(total):
    assert total % 128 == 0
    out, off = [], 0
    while off < total:
        s = min(PIECE, total - off)
        out.append((off, s))
        off += s
    return out


_mesh = plsc.VectorSubcoreMesh(core_axis_name="c", subcore_axis_name="s")


@functools.partial(
    pl.kernel,
    mesh=_mesh,
    out_type=jax.ShapeDtypeStruct((BATCH,), jnp.float32),
    scratch_types=[
        pltpu.VMEM((IDX_Q,), jnp.int32),
        pltpu.VMEM((IDX_Q,), jnp.int32),
        pltpu.VMEM((IDX_Q,), jnp.float32),
        pltpu.VMEM((IDX_Q,), jnp.float32),
        pltpu.VMEM((B_PER_W,), jnp.float32),
        pltpu.VMEM((LANES,), jnp.float32),
        pltpu.VMEM((PIECE,), jnp.float32),
        pltpu.VMEM((PIECE,), jnp.float32),
        pltpu.VMEM_SHARED((SPM_TABLE,), jnp.float32),
        pltpu.SemaphoreType.DMA,
        pltpu.SemaphoreType.DMA,
        pltpu.SemaphoreType.DMA,
        pltpu.SemaphoreType.DMA,
    ],
)
def _wide_kernel(xt_hbm, w_hbm, wtail_hbm, bias_hbm, out_hbm, i0_v, i1_v,
                 v0_v, v1_v, out_v, bias_v, b0_v, b1_v, w_spm,
                 sem_g, sem_i, sem_r, sem_s):
    cid = lax.axis_index("c")
    sid = lax.axis_index("s")
    wid = cid * NS + sid
    base_b = wid * B_PER_W

    idx_bufs = (i0_v, i1_v)
    val_bufs = (v0_v, v1_v)

    def fire_idx(q):
        dst = idx_bufs[q % 2]

        def fbody(f, _):
            pltpu.async_copy(
                xt_hbm.at[f, pl.ds(base_b + q * B_Q, B_Q)],
                dst.at[pl.ds(f * B_Q, B_Q)], sem_i)
            return 0
        lax.fori_loop(0, N_FIELDS, fbody, 0)

    def drain_idx(q):
        dst = idx_bufs[q % 2]

        def fbody(f, _):
            pltpu.make_async_copy(
                xt_hbm.at[f, pl.ds(base_b + q * B_Q, B_Q)],
                dst.at[pl.ds(f * B_Q, B_Q)], sem_i).wait()
            return 0
        lax.fori_loop(0, N_FIELDS, fbody, 0)

    pltpu.sync_copy(bias_hbm, bias_v)
    fire_idx(0)

    bounce = (b0_v, b1_v)

    def stage(base_w, pieces):
        reads, writes = {}, {}
        for k, (po, ps) in enumerate(pieces):
            b = bounce[k % 2]
            off = pl.multiple_of(base_w + po, 128)
            if k >= 2:
                oo, os_ = writes[k % 2]
                pltpu.make_async_copy(
                    b.at[pl.ds(0, os_)], w_spm.at[pl.ds(oo, os_)], sem_s).wait()
            pltpu.async_copy(w_hbm.at[0, pl.ds(off, ps)], b.at[pl.ds(0, ps)],
                             sem_r)
            reads[k % 2] = (off, ps)
            if k >= 1:
                po_, ps_ = reads[(k - 1) % 2]
                bb = bounce[(k - 1) % 2]
                pltpu.make_async_copy(
                    w_hbm.at[0, pl.ds(po_, ps_)], bb.at[pl.ds(0, ps_)],
                    sem_r).wait()
                pltpu.async_copy(bb.at[pl.ds(0, ps_)],
                                 w_spm.at[pl.ds(po_, ps_)], sem_s)
                writes[(k - 1) % 2] = (po_, ps_)
        k = len(pieces) - 1
        po_, ps_ = reads[k % 2]
        pltpu.make_async_copy(
            w_hbm.at[0, pl.ds(po_, ps_)], bounce[k % 2].at[pl.ds(0, ps_)],
            sem_r).wait()
        pltpu.async_copy(bounce[k % 2].at[pl.ds(0, ps_)],
                         w_spm.at[pl.ds(po_, ps_)], sem_s)
        writes[k % 2] = (po_, ps_)
        for kk in sorted(writes):
            oo, os_ = writes[kk]
            pltpu.make_async_copy(
                bounce[kk].at[pl.ds(0, os_)], w_spm.at[pl.ds(oo, os_)],
                sem_s).wait()

    @pl.when(sid < 15)
    def _():
        stage(sid * W_CHUNK, _pieces(W_CHUNK))

    @pl.when(sid == 15)
    def _():
        stage(LAST_BASE, _pieces(TAIL_BASE - LAST_BASE))
        pltpu.sync_copy(wtail_hbm.at[0], b0_v.at[pl.ds(0, 128)])
        pltpu.sync_copy(b0_v.at[pl.ds(0, 128)],
                        w_spm.at[pl.ds(TAIL_BASE, 128)])

    drain_idx(0)
    plsc.subcore_barrier()

    bias_vec = bias_v[...]

    H_SPLIT = 3584

    def fire_gather(q):
        ib, vb = idx_bufs[q % 2], val_bufs[q % 2]
        pltpu.async_copy(w_hbm.at[0].at[ib.at[pl.ds(0, H_SPLIT)]],
                         vb.at[pl.ds(0, H_SPLIT)], sem_r)
        pltpu.async_copy(w_spm.at[ib.at[pl.ds(H_SPLIT, IDX_Q - H_SPLIT)]],
                         vb.at[pl.ds(H_SPLIT, IDX_Q - H_SPLIT)], sem_g)

    def wait_gather(q):
        ib, vb = idx_bufs[q % 2], val_bufs[q % 2]
        pltpu.make_async_copy(w_hbm.at[0].at[ib.at[pl.ds(0, H_SPLIT)]],
                              vb.at[pl.ds(0, H_SPLIT)], sem_r).wait()
        pltpu.make_async_copy(w_spm.at[ib.at[pl.ds(H_SPLIT, IDX_Q - H_SPLIT)]],
                              vb.at[pl.ds(H_SPLIT, IDX_Q - H_SPLIT)], sem_g).wait()

    fire_gather(0)
    for q in range(N_PASS):
        if q + 1 < N_PASS:
            fire_idx(q + 1)
        wait_gather(q)
        if q + 1 < N_PASS:
            drain_idx(q + 1)
            fire_gather(q + 1)
        vals_v = val_bufs[q % 2]

        def jbody(j, _, q=q, vals_v=vals_v):
            j16 = j * LANES
            acc = bias_vec
            for f in range(N_FIELDS):
                acc = acc + vals_v[pl.ds(f * B_Q + j16, LANES)]
            out_v[pl.ds(q * B_Q + j16, LANES)] = acc
            return 0

        lax.fori_loop(0, B_Q // LANES, jbody, 0)

    pltpu.sync_copy(out_v, out_hbm.at[pl.ds(base_b, B_PER_W)])


def kernel(X, W, bias):
    xt = X.astype(jnp.int32).T
    wt = W.T
    wtail = jnp.pad(W[TAIL_BASE:], ((0, 128 - (TABLE - TAIL_BASE)), (0, 0))).T
    bias16 = jnp.broadcast_to(bias.astype(jnp.float32), (LANES,))
    out = _wide_kernel(xt, wt, wtail, bias16)
    return out.reshape(BATCH, 1)


# speedup vs baseline: 1.1175x; 1.1175x over previous
import functools

import jax
import jax.numpy as jnp
from jax import lax
from jax.experimental import pallas as pl
from jax.experimental.pallas import tpu as pltpu
from jax.experimental.pallas import tpu_sc as plsc

BATCH = 16384
N_FIELDS = 100
LANES = 16
NW = 32
NS = 16
B_PER_W = BATCH // NW
N_PASS = 4
B_Q = B_PER_W // N_PASS
IDX_Q = B_Q * N_FIELDS
TABLE = 1000001
SPM_TABLE = 1000448
W_CHUNK = 62592
LAST_BASE = 15 * W_CHUNK
PIECE = 7808
TAIL_BASE = TABLE - TABLE % 128


def _pieces(total):
    assert total % 128 == 0
    out, off = [], 0
    while off < total:
        s = min(PIECE, total - off)
        out.append((off, s))
        off += s
    return out


_mesh = plsc.VectorSubcoreMesh(core_axis_name="c", subcore_axis_name="s")


@functools.partial(
    pl.kernel,
    mesh=_mesh,
    out_type=jax.ShapeDtypeStruct((BATCH,), jnp.float32),
    scratch_types=[
        pltpu.VMEM((IDX_Q,), jnp.int32),
        pltpu.VMEM((IDX_Q,), jnp.int32),
        pltpu.VMEM((IDX_Q,), jnp.float32),
        pltpu.VMEM((IDX_Q,), jnp.float32),
        pltpu.VMEM((B_PER_W,), jnp.float32),
        pltpu.VMEM((LANES,), jnp.float32),
        pltpu.VMEM((PIECE,), jnp.float32),
        pltpu.VMEM((PIECE,), jnp.float32),
        pltpu.VMEM_SHARED((SPM_TABLE,), jnp.float32),
        pltpu.SemaphoreType.DMA,
        pltpu.SemaphoreType.DMA,
        pltpu.SemaphoreType.DMA,
        pltpu.SemaphoreType.DMA,
    ],
)
def _wide_kernel(xt_hbm, w_hbm, wtail_hbm, bias_hbm, out_hbm, i0_v, i1_v,
                 v0_v, v1_v, out_v, bias_v, b0_v, b1_v, w_spm,
                 sem_g, sem_i, sem_r, sem_s):
    cid = lax.axis_index("c")
    sid = lax.axis_index("s")
    wid = cid * NS + sid
    base_b = wid * B_PER_W

    idx_bufs = (i0_v, i1_v)
    val_bufs = (v0_v, v1_v)

    def fire_idx(q):
        dst = idx_bufs[q % 2]

        def fbody(f, _):
            pltpu.async_copy(
                xt_hbm.at[f, pl.ds(base_b + q * B_Q, B_Q)],
                dst.at[pl.ds(f * B_Q, B_Q)], sem_i)
            return 0
        lax.fori_loop(0, N_FIELDS, fbody, 0)

    def drain_idx(q):
        dst = idx_bufs[q % 2]

        def fbody(f, _):
            pltpu.make_async_copy(
                xt_hbm.at[f, pl.ds(base_b + q * B_Q, B_Q)],
                dst.at[pl.ds(f * B_Q, B_Q)], sem_i).wait()
            return 0
        lax.fori_loop(0, N_FIELDS, fbody, 0)

    pltpu.sync_copy(bias_hbm, bias_v)
    fire_idx(0)

    bounce = (b0_v, b1_v)

    def stage(base_w, pieces):
        reads, writes = {}, {}
        for k, (po, ps) in enumerate(pieces):
            b = bounce[k % 2]
            off = pl.multiple_of(base_w + po, 128)
            if k >= 2:
                oo, os_ = writes[k % 2]
                pltpu.make_async_copy(
                    b.at[pl.ds(0, os_)], w_spm.at[pl.ds(oo, os_)], sem_s).wait()
            pltpu.async_copy(w_hbm.at[0, pl.ds(off, ps)], b.at[pl.ds(0, ps)],
                             sem_r)
            reads[k % 2] = (off, ps)
            if k >= 1:
                po_, ps_ = reads[(k - 1) % 2]
                bb = bounce[(k - 1) % 2]
                pltpu.make_async_copy(
                    w_hbm.at[0, pl.ds(po_, ps_)], bb.at[pl.ds(0, ps_)],
                    sem_r).wait()
                pltpu.async_copy(bb.at[pl.ds(0, ps_)],
                                 w_spm.at[pl.ds(po_, ps_)], sem_s)
                writes[(k - 1) % 2] = (po_, ps_)
        k = len(pieces) - 1
        po_, ps_ = reads[k % 2]
        pltpu.make_async_copy(
            w_hbm.at[0, pl.ds(po_, ps_)], bounce[k % 2].at[pl.ds(0, ps_)],
            sem_r).wait()
        pltpu.async_copy(bounce[k % 2].at[pl.ds(0, ps_)],
                         w_spm.at[pl.ds(po_, ps_)], sem_s)
        writes[k % 2] = (po_, ps_)
        for kk in sorted(writes):
            oo, os_ = writes[kk]
            pltpu.make_async_copy(
                bounce[kk].at[pl.ds(0, os_)], w_spm.at[pl.ds(oo, os_)],
                sem_s).wait()

    @pl.when(sid < 15)
    def _():
        stage(sid * W_CHUNK, _pieces(W_CHUNK))

    @pl.when(sid == 15)
    def _():
        stage(LAST_BASE, _pieces(TAIL_BASE - LAST_BASE))
        pltpu.sync_copy(wtail_hbm.at[0], b0_v.at[pl.ds(0, 128)])
        pltpu.sync_copy(b0_v.at[pl.ds(0, 128)],
                        w_spm.at[pl.ds(TAIL_BASE, 128)])

    drain_idx(0)
    plsc.subcore_barrier()

    bias_vec = bias_v[...]

    H_SPLIT = 1536

    def fire_gather(q):
        ib, vb = idx_bufs[q % 2], val_bufs[q % 2]
        pltpu.async_copy(w_hbm.at[0].at[ib.at[pl.ds(0, H_SPLIT)]],
                         vb.at[pl.ds(0, H_SPLIT)], sem_r)
        pltpu.async_copy(w_spm.at[ib.at[pl.ds(H_SPLIT, IDX_Q - H_SPLIT)]],
                         vb.at[pl.ds(H_SPLIT, IDX_Q - H_SPLIT)], sem_g)

    def wait_gather(q):
        ib, vb = idx_bufs[q % 2], val_bufs[q % 2]
        pltpu.make_async_copy(w_hbm.at[0].at[ib.at[pl.ds(0, H_SPLIT)]],
                              vb.at[pl.ds(0, H_SPLIT)], sem_r).wait()
        pltpu.make_async_copy(w_spm.at[ib.at[pl.ds(H_SPLIT, IDX_Q - H_SPLIT)]],
                              vb.at[pl.ds(H_SPLIT, IDX_Q - H_SPLIT)], sem_g).wait()

    fire_gather(0)
    for q in range(N_PASS):
        if q + 1 < N_PASS:
            fire_idx(q + 1)
        wait_gather(q)
        if q + 1 < N_PASS:
            drain_idx(q + 1)
            fire_gather(q + 1)
        vals_v = val_bufs[q % 2]

        def jbody(j, _, q=q, vals_v=vals_v):
            j16 = j * LANES
            acc = bias_vec
            for f in range(N_FIELDS):
                acc = acc + vals_v[pl.ds(f * B_Q + j16, LANES)]
            out_v[pl.ds(q * B_Q + j16, LANES)] = acc
            return 0

        lax.fori_loop(0, B_Q // LANES, jbody, 0)

    pltpu.sync_copy(out_v, out_hbm.at[pl.ds(base_b, B_PER_W)])


def kernel(X, W, bias):
    xt = X.astype(jnp.int32).T
    wt = W.T
    wtail = jnp.pad(W[TAIL_BASE:], ((0, 128 - (TABLE - TAIL_BASE)), (0, 0))).T
    bias16 = jnp.broadcast_to(bias.astype(jnp.float32), (LANES,))
    out = _wide_kernel(xt, wt, wtail, bias16)
    return out.reshape(BATCH, 1)
